# baseline (speedup 1.0000x reference)
import jax
import jax.numpy as jnp
from jax import lax
from jax.experimental import pallas as pl
from jax.experimental.pallas import tpu as pltpu

_BOXES = (
    (230, 87, 397, 375), (280, 23, 404, 270), (13, 16, 424, 207),
    (64, 202, 396, 389), (57, 128, 434, 275), (7, 201, 346, 366),
    (7, 176, 321, 378), (88, 80, 328, 173), (205, 228, 297, 305),
    (9, 81, 272, 330), (215, 250, 494, 440), (143, 18, 401, 196),
    (283, 28, 461, 494), (119, 37, 326, 290), (65, 225, 241, 482),
    (57, 266, 240, 404), (156, 295, 478, 439), (23, 38, 224, 340),
    (81, 329, 261, 449), (75, 124, 477, 308), (113, 115, 207, 187),
    (230, 123, 411, 452), (159, 191, 357, 317), (204, 132, 422, 477),
    (254, 38, 499, 251), (252, 172, 508, 448), (17, 81, 227, 479),
    (330, 32, 492, 447), (145, 75, 344, 471), (332, 378, 486, 442),
    (29, 285, 319, 443), (163, 339, 493, 453),
)

_B, _H, _W = 32, 512, 512


def _tc_body(boxes_ref, img_ref, out_ref):
    i = pl.program_id(0)
    a = boxes_ref[i, 0]
    b = boxes_ref[i, 1]
    c = boxes_ref[i, 2]
    d = boxes_ref[i, 3]
    rows = lax.broadcasted_iota(jnp.int32, (_H, _W), 0)
    cols = lax.broadcasted_iota(jnp.int32, (_H, _W), 1)
    mask = (rows >= a) & (rows < c) & (cols >= b) & (cols < d)
    out_ref[0] = jnp.where(mask, 0.0, img_ref[0])


def kernel(images):
    imgs3 = images.reshape(_B, _H, _W)
    boxes = jnp.asarray(_BOXES, dtype=jnp.int32)
    out = pl.pallas_call(
        _tc_body,
        grid=(_B,),
        in_specs=[
            pl.BlockSpec(memory_space=pltpu.SMEM),
            pl.BlockSpec((1, _H, _W), lambda i: (i, 0, 0)),
        ],
        out_specs=pl.BlockSpec((1, _H, _W), lambda i: (i, 0, 0)),
        out_shape=jax.ShapeDtypeStruct((_B, _H, _W), jnp.float32),
    )(boxes, imgs3)
    return out.reshape(_B, _H, _W, 1)
